# Initial kernel scaffold; baseline (speedup 1.0000x reference)
import jax
import jax.numpy as jnp
from jax.experimental import pallas as pl


def kernel(logits, temperature, top_p, presence_penalties, frequency_penalties, repetition_penalties, prompt_token_ids, output_token_ids, top_k):
    raise NotImplementedError("write your pallas kernel here")


# speedup vs baseline: 46.9727x; 46.9727x over previous
import functools

import jax
import jax.numpy as jnp
from jax import lax
from jax.experimental import pallas as pl
from jax.experimental.pallas import tpu as pltpu
from jax.experimental.pallas import tpu_sc as plsc

_TINY = 6.103515625e-05
_SAMPLING_EPS = 1e-05
_NWORKERS = 32
_LANES = 16


def _mask_body(P, O, V, rpw, ptoks_hbm, otoks_hbm, zeros_hbm,
               pmask_hbm, omask_hbm, pidx_v, oidx_v, mask_v):
    wid = lax.axis_index("s") * 2 + lax.axis_index("c")
    ones = jnp.ones((_LANES,), jnp.int32)
    for r in range(rpw):
        row = wid * rpw + r
        pltpu.sync_copy(ptoks_hbm.at[row], pidx_v)
        pltpu.sync_copy(zeros_hbm, mask_v)

        def pbody(j, carry):
            idxs = pidx_v[pl.ds(j * _LANES, _LANES)]
            plsc.store_scatter(mask_v, [idxs], ones)
            return carry

        lax.fori_loop(0, P // _LANES, pbody, 0)
        pltpu.sync_copy(mask_v, pmask_hbm.at[row])
        pltpu.sync_copy(otoks_hbm.at[row], oidx_v)
        pltpu.sync_copy(zeros_hbm, mask_v)

        def obody(j, carry):
            idxs = oidx_v[pl.ds(j * _LANES, _LANES)]
            plsc.store_scatter(mask_v, [idxs], ones)
            return carry

        lax.fori_loop(0, O // _LANES, obody, 0)
        if O % _LANES:
            idxs = oidx_v[pl.ds(O - _LANES, _LANES)]
            plsc.store_scatter(mask_v, [idxs], ones)
        pltpu.sync_copy(mask_v, omask_hbm.at[row])


def _build_masks(prompt_token_ids, output_token_ids, V):
    B, P = prompt_token_ids.shape
    O = output_token_ids.shape[1]
    rpw = B // _NWORKERS
    zeros = jnp.zeros((V,), jnp.int32)
    mesh = plsc.VectorSubcoreMesh(core_axis_name="c", subcore_axis_name="s")
    kern = functools.partial(
        pl.kernel,
        mesh=mesh,
        out_type=(jax.ShapeDtypeStruct((B, V), jnp.int32),
                  jax.ShapeDtypeStruct((B, V), jnp.int32)),
        scratch_types=[
            pltpu.VMEM((P,), jnp.int32),
            pltpu.VMEM((O,), jnp.int32),
            pltpu.VMEM((V,), jnp.int32),
        ],
        compiler_params=pltpu.CompilerParams(needs_layout_passes=False),
    )(functools.partial(_mask_body, P, O, V, rpw))
    return kern(prompt_token_ids, output_token_ids, zeros)


def _sampler_body(V, R, tt_ref, tp_ref, pp_ref, rp_ref, tk_ref,
                  logits_ref, pm_ref, om_ref, out_ref):
    rsl = pl.ds(pl.program_id(0) * R, R)
    x = logits_ref[...]
    pm = pm_ref[...] > 0
    om = om_ref[...] > 0
    penal = jnp.where(pm | om, rp_ref[rsl, :], 1.0)
    x = jnp.where(x > 0, x / penal, x * penal)
    x = x - pp_ref[rsl, :] * om.astype(jnp.float32)
    temp = tt_ref[rsl, :]
    temp = jnp.where(temp < _SAMPLING_EPS, 1.0, temp)
    x = x / temp

    i32 = lax.bitcast_convert_type(x, jnp.int32)
    u = lax.bitcast_convert_type(x, jnp.uint32)
    ukey = jnp.where(i32 < 0, ~u, u | jnp.uint32(0x80000000))

    k = tk_ref[rsl, :]

    def kbody(i, thr):
        bit = jnp.uint32(31) - i.astype(jnp.uint32)
        cand = thr | (jnp.uint32(1) << bit)
        cnt = jnp.sum((ukey >= cand).astype(jnp.int32), axis=1,
                      keepdims=True)
        return jnp.where(cnt >= k, cand, thr)

    thr = lax.fori_loop(0, 32, kbody,
                        jnp.zeros_like(k, dtype=jnp.uint32))
    surv = ukey >= thr

    mx = jnp.max(x, axis=1, keepdims=True)
    M = jnp.maximum(mx, _TINY)
    e = jnp.exp(x - M)
    eT = jnp.exp(_TINY - M)
    es = jnp.where(surv, e, 0.0)
    cntk = jnp.sum(surv.astype(jnp.int32), axis=1, keepdims=True)
    m = (V - cntk).astype(jnp.float32)
    Z = m * eT + jnp.sum(es, axis=1, keepdims=True)
    ps = es / Z
    base = m * (eT / Z)
    tgt = 1.0 - tp_ref[rsl, :]

    def pbody(i, T):
        bit = jnp.uint32(31) - i.astype(jnp.uint32)
        cand = T | (jnp.uint32(1) << bit)
        s = jnp.sum(jnp.where(ukey <= cand, ps, 0.0), axis=1,
                    keepdims=True)
        return jnp.where(base + s <= tgt, cand, T)

    T = lax.fori_loop(0, 32, pbody, jnp.zeros_like(thr))
    kept = surv & ((ukey > T) | (x == mx))

    nk = jnp.sum(kept.astype(jnp.int32), axis=1,
                 keepdims=True).astype(jnp.float32)
    ek = jnp.where(kept, e, 0.0)
    Z2 = (V - nk) * eT + jnp.sum(ek, axis=1, keepdims=True)
    out_ref[...] = jnp.where(kept, e / Z2, eT / Z2)


def _sampler_tc(logits, pmask, omask, temperature, top_p,
                presence_penalties, repetition_penalties, top_k):
    B, V = logits.shape
    R = 8
    col = lambda a, d: a.reshape(B, 1).astype(d)
    pspec = pl.BlockSpec((B, 1), lambda i: (0, 0))
    vspec = pl.BlockSpec((R, V), lambda i: (i, 0))
    return pl.pallas_call(
        functools.partial(_sampler_body, V, R),
        grid=(B // R,),
        in_specs=[pspec, pspec, pspec, pspec, pspec,
                  vspec, vspec, vspec],
        out_specs=vspec,
        out_shape=jax.ShapeDtypeStruct((B, V), jnp.float32),
    )(col(temperature, jnp.float32), col(top_p, jnp.float32),
      col(presence_penalties, jnp.float32),
      col(repetition_penalties, jnp.float32), col(top_k, jnp.int32),
      logits, pmask, omask)


def kernel(logits, temperature, top_p, presence_penalties,
           frequency_penalties, repetition_penalties, prompt_token_ids,
           output_token_ids, top_k):
    del frequency_penalties
    logits = logits.astype(jnp.float32)
    B, V = logits.shape
    pmask_i, omask_i = _build_masks(prompt_token_ids, output_token_ids, V)
    probs = _sampler_tc(logits, pmask_i, omask_i, temperature, top_p,
                        presence_penalties, repetition_penalties, top_k)
    return probs, pmask_i.astype(jnp.bool_), omask_i.astype(jnp.bool_)
